# Initial kernel scaffold; baseline (speedup 1.0000x reference)
import jax
import jax.numpy as jnp
from jax.experimental import pallas as pl


def kernel(X, Y):
    raise NotImplementedError("write your pallas kernel here")


# speedup vs baseline: 1.0705x; 1.0705x over previous
import jax
import jax.numpy as jnp
from jax.experimental import pallas as pl


def _dsq_kernel(p_ref, q_ref, out_ref):
    p = p_ref[...]
    q = q_ref[...]
    d = jnp.abs(p - q)
    dh = d >> 10
    dl = d & 1023
    out_ref[0, :] = jnp.sum(dh * dh, axis=0, dtype=jnp.int32)
    out_ref[1, :] = jnp.sum(dh * dl, axis=0, dtype=jnp.int32)
    out_ref[2, :] = jnp.sum(dl * dl, axis=0, dtype=jnp.int32)


def kernel(X, Y):
    n = X.shape[0]
    p = jnp.argsort(X).astype(jnp.int32).reshape(1024, -1)
    q = jnp.argsort(Y).astype(jnp.int32).reshape(1024, -1)
    sums = pl.pallas_call(
        _dsq_kernel,
        out_shape=jax.ShapeDtypeStruct((3, p.shape[1]), jnp.int32),
    )(p, q)
    s = sums.astype(jnp.float64)
    S = (2.0**20) * jnp.sum(s[0]) + (2.0**11) * jnp.sum(s[1]) + jnp.sum(s[2])
    r = 1.0 - 6.0 * S / (n * (n * n - 1.0))
    return -r
